# Initial kernel scaffold; baseline (speedup 1.0000x reference)
import jax
import jax.numpy as jnp
from jax.experimental import pallas as pl


def kernel(input_x, char_table, pos_table):
    raise NotImplementedError("write your pallas kernel here")


# speedup vs baseline: 3.4031x; 3.4031x over previous
import functools

import jax
import jax.numpy as jnp
from jax import lax
from jax.experimental import pallas as pl
from jax.experimental.pallas import tpu as pltpu
from jax.experimental.pallas import tpu_sc as plsc

VOCAB, EMBED, BATCH, SEQ = 1000, 64, 4096, 200
NC, NS = 2, 16
NW = NC * NS
N = BATCH * SEQ
G = 100
NG = N // G
ROWS_W = N // NW
CHUNK = 400
GPC = CHUNK // G
NCHUNK = ROWS_W // CHUNK
LANES = 16

_mesh = plsc.VectorSubcoreMesh(
    core_axis_name="c", subcore_axis_name="s", num_cores=NC, num_subcores=NS
)


@functools.partial(
    pl.kernel,
    out_type=jax.ShapeDtypeStruct((N, EMBED), jnp.float32),
    mesh=_mesh,
    scratch_types=[
        pltpu.VMEM((GPC, G), jnp.int32),
        pltpu.VMEM((CHUNK, EMBED), jnp.float32),
        pltpu.VMEM((CHUNK, EMBED), jnp.float32),
        pltpu.SemaphoreType.DMA,
    ],
    compiler_params=pltpu.CompilerParams(use_tc_tiling_on_sc=False),
)
def _embed_kernel(idx_hbm, char_hbm, pos_hbm, out_hbm, idx_v, work_v, pos_v, sem):
    wid = lax.axis_index("s") * NC + lax.axis_index("c")
    for t in range(CHUNK // SEQ):
        pltpu.sync_copy(pos_hbm, pos_v.at[pl.ds(t * SEQ, SEQ)])

    def chunk_body(it, carry):
        row0 = wid * ROWS_W + it * CHUNK
        g0 = wid * (ROWS_W // G) + it * GPC
        pltpu.sync_copy(idx_hbm.at[pl.ds(g0, GPC)], idx_v)
        copies = [
            pltpu.async_copy(
                char_hbm.at[idx_v.at[j]], work_v.at[pl.ds(j * G, G)], sem
            )
            for j in range(GPC)
        ]
        for c in copies:
            c.wait()

        def add_body(r, c2):
            for k in range(EMBED // LANES):
                plsc.addupdate(
                    work_v.at[r, pl.ds(k * LANES, LANES)],
                    pos_v[r, pl.ds(k * LANES, LANES)],
                )
            return c2

        lax.fori_loop(0, CHUNK, add_body, 0)
        pltpu.sync_copy(work_v, out_hbm.at[pl.ds(row0, CHUNK)])
        return carry

    lax.fori_loop(0, NCHUNK, chunk_body, 0)


def kernel(input_x, char_table, pos_table):
    idx = input_x.reshape(NG, G)
    out = _embed_kernel(idx, char_table, pos_table)
    return out.reshape(BATCH, SEQ, EMBED)
